# baseline (speedup 1.0000x reference)
import functools

import jax
import jax.numpy as jnp
from jax import lax
from jax.experimental import pallas as pl
from jax.experimental.pallas import tpu as pltpu
from jax.experimental.pallas import tpu_sc as plsc

B = 16384
D = 32
CARD = 1000000
SUPER = 128
RPS = SUPER // D
SHIFT = RPS.bit_length() - 1


def _make_gather(n_cores: int, n_subcores: int):
    nw = n_cores * n_subcores
    bpw = B // nw
    mesh = plsc.VectorSubcoreMesh(core_axis_name="c", subcore_axis_name="s")

    @functools.partial(
        pl.kernel,
        mesh=mesh,
        out_type=jax.ShapeDtypeStruct((B * D,), jnp.float32),
        scratch_types=[
            pltpu.VMEM((bpw,), jnp.int32),
            pltpu.VMEM((bpw,), jnp.int32),
            pltpu.VMEM((bpw, SUPER), jnp.float32),
            pltpu.VMEM((bpw * D,), jnp.float32),
            pltpu.SemaphoreType.DMA,
        ],
        compiler_params=pltpu.CompilerParams(needs_layout_passes=False),
    )
    def gather(table_hbm, idx_hbm, out_hbm, idx_v, sidx_v, super_v, out_v, sem):
        wid = lax.axis_index("s") * n_cores + lax.axis_index("c")
        base = wid * bpw
        pltpu.sync_copy(idx_hbm.at[pl.ds(base, bpw)], idx_v)

        def sbody(k, carry):
            sl = pl.ds(k * 16, 16)
            sidx_v[sl] = lax.shift_right_logical(idx_v[sl], SHIFT)
            return carry

        lax.fori_loop(0, bpw // 16, sbody, 0)
        pltpu.async_copy(table_hbm.at[sidx_v], super_v, sem).wait()

        def xbody(k, carry):
            sl = pl.ds(k * 16, 16)
            iv = idx_v[sl]
            rows = lax.iota(jnp.int32, 16) + k * 16
            colbase = (iv & (RPS - 1)) * D
            obase = rows * D
            for c in range(D):
                v = plsc.load_gather(super_v, [rows, colbase + c])
                plsc.store_scatter(out_v, [obase + c], v)
            return carry

        lax.fori_loop(0, bpw // 16, xbody, 0)
        pltpu.sync_copy(out_v, out_hbm.at[pl.ds(base * D, bpw * D)])

    return gather


def kernel(data, col_num, emb_table):
    idx = lax.dynamic_index_in_dim(data, col_num, axis=1, keepdims=False)
    table2 = emb_table.reshape(CARD // RPS, SUPER)
    info = plsc.get_sparse_core_info()
    gather = _make_gather(info.num_cores, info.num_subcores)
    out = gather(table2, idx.astype(jnp.int32))
    return out.reshape(B, D)


# speedup vs baseline: 4.5909x; 4.5909x over previous
import functools

import jax
import jax.numpy as jnp
from jax import lax
from jax.experimental import pallas as pl
from jax.experimental.pallas import tpu as pltpu
from jax.experimental.pallas import tpu_sc as plsc

B = 16384
D = 32
CARD = 1000000


def _make_gather(n_cores: int, n_subcores: int):
    bps = B // n_subcores
    cpc = D // n_cores
    mesh = plsc.VectorSubcoreMesh(core_axis_name="c", subcore_axis_name="s")

    @functools.partial(
        pl.kernel,
        mesh=mesh,
        out_type=jax.ShapeDtypeStruct((D, B), jnp.float32),
        scratch_types=[
            pltpu.VMEM((bps,), jnp.int32),
            pltpu.VMEM((bps,), jnp.float32),
            pltpu.VMEM_SHARED((CARD,), jnp.float32),
            pltpu.SemaphoreType.DMA,
        ],
    )
    def gather(tableT_hbm, idx_hbm, out_hbm, idx_v, g_v, slab, sem):
        core = lax.axis_index("c")
        sid = lax.axis_index("s")
        pltpu.sync_copy(idx_hbm.at[pl.ds(sid * bps, bps)], idx_v)
        for c_local in range(cpc):
            c_abs = core * cpc + c_local

            @pl.when(sid == 0)
            def _():
                pltpu.sync_copy(tableT_hbm.at[c_abs], slab)

            plsc.subcore_barrier()
            pltpu.async_copy(slab.at[idx_v], g_v, sem).wait()
            pltpu.sync_copy(g_v, out_hbm.at[c_abs, pl.ds(sid * bps, bps)])
            plsc.subcore_barrier()

    return gather


def kernel(data, col_num, emb_table):
    idx = lax.dynamic_index_in_dim(data, col_num, axis=1, keepdims=False)
    info = plsc.get_sparse_core_info()
    gather = _make_gather(info.num_cores, info.num_subcores)
    out = gather(emb_table.T, idx.astype(jnp.int32))
    return out.T


# speedup vs baseline: 5.4657x; 1.1906x over previous
import functools

import jax
import jax.numpy as jnp
from jax import lax
from jax.experimental import pallas as pl
from jax.experimental.pallas import tpu as pltpu
from jax.experimental.pallas import tpu_sc as plsc

B = 16384
D = 32
CARD = 1000000


def _make_gather(n_cores: int, n_subcores: int):
    bps = B // n_subcores
    cpc = D // n_cores
    mesh = plsc.VectorSubcoreMesh(core_axis_name="c", subcore_axis_name="s")

    @functools.partial(
        pl.kernel,
        mesh=mesh,
        out_type=jax.ShapeDtypeStruct((D, B), jnp.float32),
        scratch_types=[
            pltpu.VMEM((bps,), jnp.int32),
            pltpu.VMEM((bps,), jnp.float32),
            pltpu.VMEM((bps,), jnp.float32),
            pltpu.VMEM_SHARED((CARD,), jnp.float32),
            pltpu.VMEM_SHARED((CARD,), jnp.float32),
            pltpu.SemaphoreType.DMA,
            pltpu.SemaphoreType.DMA,
            pltpu.SemaphoreType.DMA,
            pltpu.SemaphoreType.DMA,
        ],
    )
    def gather(
        tableT_hbm,
        idx_hbm,
        out_hbm,
        idx_v,
        g0,
        g1,
        slab0,
        slab1,
        sem_s0,
        sem_s1,
        sem_w0,
        sem_w1,
    ):
        core = lax.axis_index("c")
        sid = lax.axis_index("s")
        pltpu.sync_copy(idx_hbm.at[pl.ds(sid * bps, bps)], idx_v)
        slabs = (slab0, slab1)
        sems = (sem_s0, sem_s1)
        gbufs = (g0, g1)
        wsems = (sem_w0, sem_w1)

        def stream(c_local, slab, sem):
            pltpu.async_copy(tableT_hbm.at[core * cpc + c_local], slab, sem)

        for p in range(2):

            @pl.when(sid == p)
            def _():
                stream(p, slabs[p], sems[p])

        for c_local in range(cpc):
            p = c_local % 2
            c_abs = core * cpc + c_local

            @pl.when(sid == p)
            def _():
                pltpu.make_async_copy(
                    tableT_hbm.at[c_abs], slabs[p], sems[p]
                ).wait()

            plsc.subcore_barrier()
            if c_local >= 2:
                pltpu.make_async_copy(
                    gbufs[p], out_hbm.at[c_abs, pl.ds(sid * bps, bps)], wsems[p]
                ).wait()
            pltpu.async_copy(slabs[p].at[idx_v], gbufs[p], sems[p]).wait()
            pltpu.async_copy(
                gbufs[p], out_hbm.at[c_abs, pl.ds(sid * bps, bps)], wsems[p]
            )
            plsc.subcore_barrier()
            if c_local + 2 < cpc:

                @pl.when(sid == p)
                def _():
                    stream(c_local + 2, slabs[p], sems[p])

        for c_local in (cpc - 2, cpc - 1):
            p = c_local % 2
            c_abs = core * cpc + c_local
            pltpu.make_async_copy(
                gbufs[p], out_hbm.at[c_abs, pl.ds(sid * bps, bps)], wsems[p]
            ).wait()

    return gather


def kernel(data, col_num, emb_table):
    idx = lax.dynamic_index_in_dim(data, col_num, axis=1, keepdims=False)
    info = plsc.get_sparse_core_info()
    gather = _make_gather(info.num_cores, info.num_subcores)
    out = gather(emb_table.T, idx.astype(jnp.int32))
    return out.T


# speedup vs baseline: 5.4678x; 1.0004x over previous
import functools

import jax
import jax.numpy as jnp
from jax import lax
from jax.experimental import pallas as pl
from jax.experimental.pallas import tpu as pltpu
from jax.experimental.pallas import tpu_sc as plsc

B = 16384
D = 32
CARD = 1000000


def _make_gather(n_cores: int, n_subcores: int):
    bps = B // n_subcores
    cpc = D // n_cores
    mesh = plsc.VectorSubcoreMesh(core_axis_name="c", subcore_axis_name="s")

    @functools.partial(
        pl.kernel,
        mesh=mesh,
        out_type=jax.ShapeDtypeStruct((D, B), jnp.float32),
        scratch_types=[
            pltpu.VMEM((bps,), jnp.int32),
            pltpu.VMEM((bps,), jnp.float32),
            pltpu.VMEM((bps,), jnp.float32),
            pltpu.VMEM_SHARED((CARD,), jnp.float32),
            pltpu.VMEM_SHARED((CARD,), jnp.float32),
            pltpu.SemaphoreType.DMA,
            pltpu.SemaphoreType.DMA,
            pltpu.SemaphoreType.DMA,
            pltpu.SemaphoreType.DMA,
        ],
    )
    def gather(
        tableT_hbm,
        idx_hbm,
        out_hbm,
        idx_v,
        g0,
        g1,
        slab0,
        slab1,
        sem_s0,
        sem_s1,
        sem_w0,
        sem_w1,
    ):
        core = lax.axis_index("c")
        sid = lax.axis_index("s")
        pltpu.sync_copy(idx_hbm.at[pl.ds(sid * bps, bps)], idx_v)
        slabs = (slab0, slab1)
        sems = (sem_s0, sem_s1)
        gbufs = (g0, g1)
        wsems = (sem_w0, sem_w1)
        order = [(c % 2) * (cpc // 2) + (c // 2) for c in range(cpc)]

        def stream(c_local, slab, sem):
            pltpu.async_copy(tableT_hbm.at[core * cpc + c_local], slab, sem)

        for p in range(2):

            @pl.when(sid == p)
            def _():
                stream(order[p], slabs[p], sems[p])

        for step in range(cpc):
            p = step % 2
            c_local = order[step]
            c_abs = core * cpc + c_local

            @pl.when(sid == p)
            def _():
                pltpu.make_async_copy(
                    tableT_hbm.at[c_abs], slabs[p], sems[p]
                ).wait()

            plsc.subcore_barrier()
            if step >= 2:
                pltpu.make_async_copy(
                    gbufs[p], out_hbm.at[c_abs, pl.ds(sid * bps, bps)], wsems[p]
                ).wait()
            pltpu.async_copy(slabs[p].at[idx_v], gbufs[p], sems[p]).wait()
            pltpu.async_copy(
                gbufs[p], out_hbm.at[c_abs, pl.ds(sid * bps, bps)], wsems[p]
            )
            plsc.subcore_barrier()
            if step + 2 < cpc:

                @pl.when(sid == p)
                def _():
                    stream(order[step + 2], slabs[p], sems[p])

        for step in (cpc - 2, cpc - 1):
            p = step % 2
            c_abs = core * cpc + order[step]
            pltpu.make_async_copy(
                gbufs[p], out_hbm.at[c_abs, pl.ds(sid * bps, bps)], wsems[p]
            ).wait()

    return gather


def kernel(data, col_num, emb_table):
    idx = lax.dynamic_index_in_dim(data, col_num, axis=1, keepdims=False)
    info = plsc.get_sparse_core_info()
    gather = _make_gather(info.num_cores, info.num_subcores)
    out = gather(emb_table.T, idx.astype(jnp.int32))
    return out.T
